# Initial kernel scaffold; baseline (speedup 1.0000x reference)
import jax
import jax.numpy as jnp
from jax.experimental import pallas as pl


def kernel(input_ids, token_embeddings_weight, position_embeddings, cls_embedding):
    raise NotImplementedError("write your pallas kernel here")


# speedup vs baseline: 3.3761x; 3.3761x over previous
import functools

import jax
import jax.numpy as jnp
from jax import lax
from jax.experimental import pallas as pl
from jax.experimental.pallas import tpu as pltpu
from jax.experimental.pallas import tpu_sc as plsc

B = 4096
S = 200
P = 201
D = 64
NW = 32
ROWS_PER_W = B // NW
CHUNK = 100
NCHUNK = S // CHUNK

_mesh = plsc.VectorSubcoreMesh(core_axis_name="c", subcore_axis_name="s")


@functools.partial(
    pl.kernel,
    mesh=_mesh,
    out_type=jax.ShapeDtypeStruct((B, P, D), jnp.float32),
    scratch_types=[
        pltpu.VMEM((ROWS_PER_W, NCHUNK, CHUNK), jnp.int32),
        pltpu.VMEM((P, D), jnp.float32),
        pltpu.VMEM((D,), jnp.float32),
        pltpu.VMEM((P, D), jnp.float32),
        pltpu.SemaphoreType.DMA,
    ],
    compiler_params=pltpu.CompilerParams(use_tc_tiling_on_sc=False),
)
def _sc_embed(ids_hbm, table_hbm, pos_hbm, cls_hbm, out_hbm,
              idx_v, pos_v, cls_v, g0, gsem):
    wid = lax.axis_index("s") * 2 + lax.axis_index("c")
    base = wid * ROWS_PER_W

    pltpu.sync_copy(ids_hbm.at[pl.ds(base, ROWS_PER_W)], idx_v)
    pltpu.sync_copy(pos_hbm, pos_v)
    pltpu.sync_copy(cls_hbm, cls_v)

    for k in range(D // 16):
        sl = pl.ds(16 * k, 16)
        g0[S, sl] = cls_v[sl] + pos_v[S, sl]

    def per_row(b, carry):
        cp0 = pltpu.async_copy(
            table_hbm.at[idx_v.at[b, 0]], g0.at[pl.ds(0, CHUNK)], gsem)
        cp1 = pltpu.async_copy(
            table_hbm.at[idx_v.at[b, 1]], g0.at[pl.ds(CHUNK, CHUNK)], gsem)
        cp0.wait()
        cp1.wait()

        def add_row(i, c2):
            for k in range(D // 16):
                sl = pl.ds(16 * k, 16)
                plsc.addupdate(g0.at[i, sl], pos_v[i, sl])
            return c2

        lax.fori_loop(0, S, add_row, 0)
        pltpu.sync_copy(g0, out_hbm.at[base + b])
        return carry

    lax.fori_loop(0, ROWS_PER_W, per_row, 0)


@jax.jit
def kernel(input_ids, token_embeddings_weight, position_embeddings,
           cls_embedding):
    ids3 = input_ids.reshape(B, NCHUNK, CHUNK)
    return _sc_embed(ids3, token_embeddings_weight, position_embeddings,
                     cls_embedding)


# speedup vs baseline: 4.2374x; 1.2551x over previous
import functools

import jax
import jax.numpy as jnp
from jax import lax
from jax.experimental import pallas as pl
from jax.experimental.pallas import tpu as pltpu
from jax.experimental.pallas import tpu_sc as plsc

B = 4096
S = 200
P = 201
D = 64
NW = 32
ROWS_PER_W = B // NW
CHUNK = 100
NCHUNK = S // CHUNK
NBUF = 3

_mesh = plsc.VectorSubcoreMesh(core_axis_name="c", subcore_axis_name="s")


@functools.partial(
    pl.kernel,
    mesh=_mesh,
    out_type=jax.ShapeDtypeStruct((B, P, D), jnp.float32),
    scratch_types=[
        pltpu.VMEM((ROWS_PER_W, NCHUNK, CHUNK), jnp.int32),
        pltpu.VMEM((P, D), jnp.float32),
        pltpu.VMEM((D,), jnp.float32),
        [pltpu.VMEM((P, D), jnp.float32) for _ in range(NBUF)],
        [pltpu.SemaphoreType.DMA for _ in range(NBUF)],
        [pltpu.SemaphoreType.DMA for _ in range(NBUF)],
    ],
    compiler_params=pltpu.CompilerParams(use_tc_tiling_on_sc=False),
)
def _sc_embed(ids_hbm, table_hbm, pos_hbm, cls_hbm, out_hbm,
              idx_v, pos_v, cls_v, bufs, gsems, ssems):
    wid = lax.axis_index("s") * 2 + lax.axis_index("c")
    base = wid * ROWS_PER_W

    pltpu.sync_copy(ids_hbm.at[pl.ds(base, ROWS_PER_W)], idx_v)
    pltpu.sync_copy(pos_hbm, pos_v)
    pltpu.sync_copy(cls_hbm, cls_v)

    for k in range(D // 16):
        sl = pl.ds(16 * k, 16)
        v = cls_v[sl] + pos_v[S, sl]
        for s in range(NBUF):
            bufs[s][S, sl] = v

    def fire_gather(b, s):
        for j in range(NCHUNK):
            pltpu.async_copy(table_hbm.at[idx_v.at[b, j]],
                             bufs[s].at[pl.ds(j * CHUNK, CHUNK)], gsems[s])

    def wait_gather(b, s):
        for j in range(NCHUNK):
            pltpu.make_async_copy(table_hbm.at[idx_v.at[b, j]],
                                  bufs[s].at[pl.ds(j * CHUNK, CHUNK)],
                                  gsems[s]).wait()

    def wait_store(s):
        pltpu.make_async_copy(bufs[s], out_hbm.at[base], ssems[s]).wait()

    def add_pos(s):
        g = bufs[s]

        def add_row(i, c):
            for k in range(D // 16):
                sl = pl.ds(16 * k, 16)
                plsc.addupdate(g.at[i, sl], pos_v[i, sl])
            return c

        lax.fori_loop(0, S, add_row, 0)

    def finish_row(b, s):
        wait_gather(b, s)
        add_pos(s)
        pltpu.async_copy(bufs[s], out_hbm.at[base + b], ssems[s])

    fire_gather(0, 0)
    fire_gather(1, 1)
    finish_row(0, 0)
    fire_gather(2, 2)

    def body(i, carry):
        for s_off in range(NBUF):
            s = (1 + s_off) % NBUF
            r = NBUF * i + 1 + s_off
            finish_row(r, s)
            s2 = (s + 2) % NBUF

            @pl.when(r + 2 < ROWS_PER_W)
            def _():
                wait_store(s2)
                fire_gather(r + 2, s2)

        return carry

    lax.fori_loop(0, (ROWS_PER_W - 2) // NBUF, body, 0)

    finish_row(ROWS_PER_W - 1, (ROWS_PER_W - 1) % NBUF)
    for s in range(NBUF):
        wait_store(s)


@jax.jit
def kernel(input_ids, token_embeddings_weight, position_embeddings,
           cls_embedding):
    ids3 = input_ids.reshape(B, NCHUNK, CHUNK)
    return _sc_embed(ids3, token_embeddings_weight, position_embeddings,
                     cls_embedding)
